# baseline (speedup 1.0000x reference)
import functools

import jax
import jax.numpy as jnp
from jax import lax
from jax.experimental import pallas as pl
from jax.experimental.pallas import tpu as pltpu
from jax.experimental.pallas import tpu_sc as plsc

K = 12
DOUT = 256
B = 16384
N = B * 32
NV = K * 64
NT = NV // 128
NC, NS = 2, 16
NW = NC * NS
BPW = B // NW
CBAGS = 128
NCHUNK = BPW // CBAGS
NPACK = 4
CW = CBAGS // NPACK
CHWORDS = NT * CW * 128
VSTAGE = CBAGS * 32 + 16
TSTAGE = 4096


def _build_w2(pieces, ranks, files, tiles, mask):
    def body(p_ref, r_ref, f_ref, t_ref, m_ref, o_ref):
        merged = t_ref[...] + (p_ref[...] + r_ref[...] + f_ref[...]) * m_ref[...]
        o_ref[:, :DOUT] = merged.reshape(NV, DOUT).astype(jnp.bfloat16)
        rolled = jnp.concatenate([merged[K // 2:], merged[:K // 2]], axis=0)
        flipped = jnp.concatenate(
            [rolled[:, 7 - i:8 - i] for i in range(8)], axis=1)
        o_ref[:, DOUT:] = flipped.reshape(NV, DOUT).astype(jnp.bfloat16)

    return pl.pallas_call(
        body,
        out_shape=jax.ShapeDtypeStruct((NV, 2 * DOUT), jnp.bfloat16),
    )(pieces, ranks, files, tiles, mask)


def _sc_hist(values_padded, lengths, zeros_hbm_op):
    mesh = plsc.VectorSubcoreMesh(core_axis_name="c", subcore_axis_name="s")

    @functools.partial(
        pl.kernel,
        out_type=[
            jax.ShapeDtypeStruct((NT * (B // NPACK) * 128,), jnp.int32),
            jax.ShapeDtypeStruct((NW * NV,), jnp.float32),
        ],
        mesh=mesh,
        compiler_params=pltpu.CompilerParams(needs_layout_passes=False),
        scratch_types=[
            pltpu.VMEM((B + 16,), jnp.int32),
            pltpu.VMEM((VSTAGE,), jnp.int32),
            pltpu.VMEM((VSTAGE,), jnp.int32),
            pltpu.VMEM((CHWORDS,), jnp.int32),
            pltpu.VMEM((CHWORDS,), jnp.int32),
            pltpu.VMEM((NV,), jnp.float32),
            pltpu.VMEM((TSTAGE + 16,), jnp.int32),
            pltpu.VMEM((TSTAGE + 16,), jnp.int32),
            pltpu.SemaphoreType.DMA,
            pltpu.SemaphoreType.DMA,
            pltpu.SemaphoreType.DMA,
            pltpu.SemaphoreType.DMA,
            pltpu.SemaphoreType.DMA,
            pltpu.SemaphoreType.DMA,
            pltpu.SemaphoreType.DMA,
            pltpu.SemaphoreType.DMA,
        ],
    )
    def k(values_hbm, lengths_hbm, zeros_hbm, hist_hbm, tail_hbm,
          len_v, vals_v0, vals_v1, hist_v0, hist_v1, tacc_v,
          tstage_v0, tstage_v1,
          hsem0, hsem1, vsem0, vsem1, zsem0, zsem1, tsem0, tsem1):
        hsems = (hsem0, hsem1)
        vsems = (vsem0, vsem1)
        zsems = (zsem0, zsem1)
        tsems = (tsem0, tsem1)
        vals_bufs = (vals_v0, vals_v1)
        hist_bufs = (hist_v0, hist_v1)
        tstage_bufs = (tstage_v0, tstage_v1)
        wid = lax.axis_index("s") * NC + lax.axis_index("c")
        lane = lax.iota(jnp.int32, 16)
        onesf = jnp.ones((16,), jnp.float32)
        zeros16 = jnp.zeros((16,), jnp.int32)

        pltpu.sync_copy(lengths_hbm, len_v.at[pl.ds(0, B)])

        my_first = wid * BPW

        def pre_body(q, carry):
            tot_vec, base_vec = carry
            s = len_v[pl.ds(q * 64, 16)]
            for u in range(1, 4):
                s = s + len_v[pl.ds(q * 64 + u * 16, 16)]
            inc = (q * 64 < my_first).astype(jnp.int32)
            return tot_vec + s, base_vec + s * inc

        with jax.named_scope("sc_prefix"):
            tot_vec, base_vec = lax.fori_loop(0, B // 64, pre_body,
                                              (zeros16, zeros16))

        def _hsum(v):
            s = v[0]
            for i in range(1, 16):
                s = s + v[i]
            return s

        total = _hsum(tot_vec)
        base = _hsum(base_vec)

        def _span_sum(c):
            def body(q, acc):
                return acc + len_v[pl.ds(my_first + c * CBAGS + q * 16, 16)]
            return _hsum(lax.fori_loop(0, CBAGS // 16, body, zeros16))

        offs = [base]
        for c in range(NCHUNK - 1):
            offs.append(offs[-1] + _span_sum(c))

        vdescs = [None] * NCHUNK
        vbase = [None] * NCHUNK

        def _fire_stage(c):
            vbase[c] = jnp.minimum((offs[c] // 8) * 8, jnp.int32(N - VSTAGE))
            vdescs[c] = pltpu.async_copy(
                values_hbm.at[pl.ds(vbase[c], VSTAGE)],
                vals_bufs[c % 2], vsems[c % 2])

        _fire_stage(0)
        _fire_stage(1)

        zdescs = [None] * NCHUNK

        def _fire_zero(c):
            zdescs[c] = pltpu.async_copy(
                zeros_hbm, hist_bufs[c % 2], zsems[c % 2])

        _fire_zero(0)
        _fire_zero(1)

        n_tail = N - total
        gpw = (n_tail + 16 * NW - 1) // (16 * NW)
        start_w = total + wid * gpw * 16
        span = gpw * 16
        limit = jnp.minimum(jnp.int32(N), start_w + span)
        n_out = (span + TSTAGE - 1) // TSTAGE
        NTO = 4
        tdescs = [None] * NTO
        tst_al = [None] * NTO

        def _fire_tail(o):
            st = start_w + o * TSTAGE
            tst_al[o] = jnp.minimum((st // 8) * 8,
                                    jnp.int32(N - (TSTAGE + 16)))
            tdescs[o] = pltpu.async_copy(
                values_hbm.at[pl.ds(tst_al[o], TSTAGE + 16)],
                tstage_bufs[o % 2], tsems[o % 2])

        _fire_tail(0)
        _fire_tail(1)

        hdescs = [None] * NCHUNK
        for c in range(NCHUNK):
            buf = c % 2
            histb = hist_bufs[buf]
            valsb = vals_bufs[buf]

            with jax.named_scope("sc_stagewait"):
                vdescs[c].wait()
                zdescs[c].wait()

            lo0 = offs[c] - vbase[c]

            def bag_group(g, o, histb=histb, valsb=valsb, lo0=lo0, c=c):
                lv = len_v[pl.ds(my_first + c * CBAGS + g * 16, 16)]
                amt = zeros16 + (jnp.int32(1) << ((g >> 1) * 8))
                gbase = (g & 1) * (16 * 128)
                for u in range(16):
                    ln = lv[u]
                    lo = o + lo0
                    jb = gbase + u * 128

                    def scat(v, m, jb=jb):
                        idx = ((v >> 7) * (CW * 128)) + jb + (v & 127)
                        plsc.addupdate_scatter(histb, [idx], amt, mask=m)

                    m0 = lane < ln
                    v0 = plsc.load_gather(valsb, [lo + lane], mask=m0)
                    scat(v0, m0 & (v0 < NV))

                    @pl.when(ln > 16)
                    def _(lo=lo, ln=ln, scat=scat):
                        m1 = lane + 16 < ln
                        v1 = plsc.load_gather(valsb, [lo + 16 + lane],
                                              mask=m1)
                        scat(v1, m1 & (v1 < NV))

                    o = o + ln
                return o

            with jax.named_scope("sc_bags"):
                lax.fori_loop(0, CBAGS // 16, bag_group, jnp.int32(0))

            if c + 2 < NCHUNK:
                _fire_stage(c + 2)

            row0 = my_first + c * CBAGS
            with jax.named_scope("sc_histout"):
                hdescs[c] = []
                for vt in range(NT):
                    hdescs[c].append(pltpu.async_copy(
                        histb.at[pl.ds(vt * CW * 128, CW * 128)],
                        hist_hbm.at[pl.ds(vt * (B // NPACK) * 128
                                          + (row0 // NPACK) * 128, CW * 128)],
                        hsems[buf]))

            if c + 1 < NCHUNK and c >= 1:
                with jax.named_scope("sc_histdrain"):
                    for d in hdescs[c - 1]:
                        d.wait()
                _fire_zero(c + 1)

        def tz(z, _):
            tacc_v[pl.ds(z * 16, 16)] = jnp.zeros((16,), jnp.float32)
            return 0
        lax.fori_loop(0, NV // 16, tz, 0)

        with jax.named_scope("sc_tail"):
            for o in range(NTO):
                tdescs[o].wait()
                st = start_w + o * TSTAGE
                shift = st - tst_al[o]
                tbuf = tstage_bufs[o % 2]

                @pl.when(o < n_out)
                def _(st=st, shift=shift, tbuf=tbuf):
                    def tinner(g, _):
                        for u in range(4):
                            pos = st + g * 64 + u * 16
                            mp = (pos + lane) < limit
                            vv = plsc.load_gather(
                                tbuf, [shift + g * 64 + u * 16 + lane],
                                mask=mp)
                            plsc.addupdate_scatter(tacc_v, [vv], onesf,
                                                   mask=mp & (vv < NV))
                        return 0

                    lax.fori_loop(0, TSTAGE // 64, tinner, 0)

                if o + 2 < NTO:
                    _fire_tail(o + 2)

            for vt in range(NT):
                pltpu.sync_copy(
                    tacc_v.at[pl.ds(vt * 128, 128)],
                    tail_hbm.at[pl.ds(vt * NW * 128 + wid * 128, 128)])

        with jax.named_scope("sc_finaldrain"):
            for c in (NCHUNK - 2, NCHUNK - 1):
                for d in hdescs[c]:
                    d.wait()

    return k(values_padded, lengths, zeros_hbm_op)


def _matmul(hist3p, w2, tail):
    BM = 1024
    BMP = BM // NPACK
    nb = B // BM

    def body(h_ref, w2_ref, t_ref, oa_ref, ob_ref, hf_ref):
        for j in range(NT):
            w = h_ref[j]
            bytes_f = [((w >> (8 * kk)) & 0xFF).astype(jnp.bfloat16)
                       for kk in range(NPACK)]
            for g in range(BM // 128):
                for kk in range(NPACK):
                    hf_ref[g * 128 + kk * 32:g * 128 + (kk + 1) * 32,
                           j * 128:(j + 1) * 128] = (
                        bytes_f[kk][g * 32:(g + 1) * 32])
        acc = jnp.dot(hf_ref[...], w2_ref[...],
                      preferred_element_type=jnp.float32)
        tcon = jnp.dot(
            jnp.sum(t_ref[0], axis=0, keepdims=True).astype(jnp.bfloat16),
            w2_ref[:128, :], preferred_element_type=jnp.float32)
        for j in range(1, NT):
            tcon = tcon + jnp.dot(
                jnp.sum(t_ref[j], axis=0, keepdims=True).astype(jnp.bfloat16),
                w2_ref[j * 128:(j + 1) * 128, :],
                preferred_element_type=jnp.float32)
        row = lax.broadcasted_iota(jnp.int32, (BM, 1), 0)
        sel = (row == BM - 1) & (pl.program_id(0) == nb - 1)
        acc = jnp.clip(acc + jnp.where(sel, tcon, 0.0), 0.0, 1.0)
        oa_ref[...] = acc[:, :DOUT]
        ob_ref[...] = acc[:, DOUT:]

    return pl.pallas_call(
        body,
        grid=(nb,),
        in_specs=[
            pl.BlockSpec((NT, BMP, 128), lambda i: (0, i, 0)),
            pl.BlockSpec((NV, 2 * DOUT), lambda i: (0, 0)),
            pl.BlockSpec((NT, NW, 128), lambda i: (0, 0, 0)),
        ],
        out_specs=[
            pl.BlockSpec((BM, DOUT), lambda i: (i, 0)),
            pl.BlockSpec((BM, DOUT), lambda i: (i, 0)),
        ],
        out_shape=[
            jax.ShapeDtypeStruct((B, DOUT), jnp.float32),
            jax.ShapeDtypeStruct((B, DOUT), jnp.float32),
        ],
        scratch_shapes=[pltpu.VMEM((BM, NV), jnp.bfloat16)],
        compiler_params=pltpu.CompilerParams(
            dimension_semantics=("parallel",)),
    )(hist3p, w2, tail)


def kernel(pieces, ranks, files, tiles, factorization_mask, values, lengths):
    w2 = _build_w2(pieces, ranks, files, tiles, factorization_mask)
    zeros_op = jnp.zeros((CHWORDS,), jnp.int32)
    hist_flat, tail_flat = _sc_hist(values, lengths, zeros_op)
    hist3p = hist_flat.reshape(NT, B // NPACK, 128)
    tail3 = tail_flat.reshape(NT, NW, 128)
    return _matmul(hist3p, w2, tail3)


# speedup vs baseline: 1.1922x; 1.1922x over previous
import functools

import jax
import jax.numpy as jnp
from jax import lax
from jax.experimental import pallas as pl
from jax.experimental.pallas import tpu as pltpu
from jax.experimental.pallas import tpu_sc as plsc

K = 12
DOUT = 256
B = 16384
N = B * 32
NV = K * 64
NT = NV // 128
NC, NS = 2, 16
NW = NC * NS
BPW = B // NW
CBAGS = 128
NCHUNK = BPW // CBAGS
NPACK = 4
CW = CBAGS // NPACK
CHWORDS = NT * CW * 128
VSTAGE = CBAGS * 32 + 16
TSTAGE = 4096


def _build_w2(pieces, ranks, files, tiles, mask):
    def body(p_ref, r_ref, f_ref, t_ref, m_ref, o_ref):
        merged = t_ref[...] + (p_ref[...] + r_ref[...] + f_ref[...]) * m_ref[...]
        o_ref[:, :DOUT] = merged.reshape(NV, DOUT).astype(jnp.bfloat16)
        rolled = jnp.concatenate([merged[K // 2:], merged[:K // 2]], axis=0)
        flipped = jnp.concatenate(
            [rolled[:, 7 - i:8 - i] for i in range(8)], axis=1)
        o_ref[:, DOUT:] = flipped.reshape(NV, DOUT).astype(jnp.bfloat16)

    return pl.pallas_call(
        body,
        out_shape=jax.ShapeDtypeStruct((NV, 2 * DOUT), jnp.bfloat16),
    )(pieces, ranks, files, tiles, mask)


def _sc_hist(values_padded, lengths):
    mesh = plsc.VectorSubcoreMesh(core_axis_name="c", subcore_axis_name="s")

    @functools.partial(
        pl.kernel,
        out_type=[
            jax.ShapeDtypeStruct((NT * (B // NPACK) * 128,), jnp.int32),
            jax.ShapeDtypeStruct((NW * NV,), jnp.float32),
        ],
        mesh=mesh,
        compiler_params=pltpu.CompilerParams(needs_layout_passes=False),
        scratch_types=[
            pltpu.VMEM((B + 16,), jnp.int32),
            pltpu.VMEM((VSTAGE,), jnp.int32),
            pltpu.VMEM((VSTAGE,), jnp.int32),
            pltpu.VMEM((CHWORDS,), jnp.int32),
            pltpu.VMEM((CHWORDS,), jnp.int32),
            pltpu.VMEM((NV,), jnp.float32),
            pltpu.VMEM((TSTAGE + 16,), jnp.int32),
            pltpu.VMEM((TSTAGE + 16,), jnp.int32),
            pltpu.SemaphoreType.DMA,
            pltpu.SemaphoreType.DMA,
            pltpu.SemaphoreType.DMA,
            pltpu.SemaphoreType.DMA,
            pltpu.SemaphoreType.DMA,
            pltpu.SemaphoreType.DMA,
        ],
    )
    def k(values_hbm, lengths_hbm, hist_hbm, tail_hbm,
          len_v, vals_v0, vals_v1, hist_v0, hist_v1, tacc_v,
          tstage_v0, tstage_v1,
          hsem0, hsem1, vsem0, vsem1, tsem0, tsem1):
        hsems = (hsem0, hsem1)
        vsems = (vsem0, vsem1)
        tsems = (tsem0, tsem1)
        vals_bufs = (vals_v0, vals_v1)
        hist_bufs = (hist_v0, hist_v1)
        tstage_bufs = (tstage_v0, tstage_v1)
        wid = lax.axis_index("s") * NC + lax.axis_index("c")
        lane = lax.iota(jnp.int32, 16)
        onesf = jnp.ones((16,), jnp.float32)
        zeros16 = jnp.zeros((16,), jnp.int32)

        pltpu.sync_copy(lengths_hbm, len_v.at[pl.ds(0, B)])

        my_first = wid * BPW

        def pre_body(q, carry):
            tot_vec, base_vec = carry
            s = len_v[pl.ds(q * 64, 16)]
            for u in range(1, 4):
                s = s + len_v[pl.ds(q * 64 + u * 16, 16)]
            inc = (q * 64 < my_first).astype(jnp.int32)
            return tot_vec + s, base_vec + s * inc

        with jax.named_scope("sc_prefix"):
            tot_vec, base_vec = lax.fori_loop(0, B // 64, pre_body,
                                              (zeros16, zeros16))

        def _hsum(v):
            s = v[0]
            for i in range(1, 16):
                s = s + v[i]
            return s

        total = _hsum(tot_vec)
        base = _hsum(base_vec)

        def _span_sum(c):
            def body(q, acc):
                return acc + len_v[pl.ds(my_first + c * CBAGS + q * 16, 16)]
            return _hsum(lax.fori_loop(0, CBAGS // 16, body, zeros16))

        offs = [base]
        for c in range(NCHUNK - 1):
            offs.append(offs[-1] + _span_sum(c))

        vdescs = [None] * NCHUNK
        vbase = [None] * NCHUNK

        def _fire_stage(c):
            vbase[c] = jnp.minimum((offs[c] // 8) * 8, jnp.int32(N - VSTAGE))
            vdescs[c] = pltpu.async_copy(
                values_hbm.at[pl.ds(vbase[c], VSTAGE)],
                vals_bufs[c % 2], vsems[c % 2])

        _fire_stage(0)
        _fire_stage(1)

        n_tail = N - total
        gpw = (n_tail + 16 * NW - 1) // (16 * NW)
        start_w = total + wid * gpw * 16
        span = gpw * 16
        limit = jnp.minimum(jnp.int32(N), start_w + span)
        n_out = (span + TSTAGE - 1) // TSTAGE
        NTO = 4
        tdescs = [None] * NTO
        tst_al = [None] * NTO

        def _fire_tail(o):
            st = start_w + o * TSTAGE
            tst_al[o] = jnp.minimum((st // 8) * 8,
                                    jnp.int32(N - (TSTAGE + 16)))
            tdescs[o] = pltpu.async_copy(
                values_hbm.at[pl.ds(tst_al[o], TSTAGE + 16)],
                tstage_bufs[o % 2], tsems[o % 2])

        _fire_tail(0)
        _fire_tail(1)

        hdescs = [None] * NCHUNK
        for c in range(NCHUNK):
            buf = c % 2
            histb = hist_bufs[buf]
            valsb = vals_bufs[buf]

            def zbody(z, _, histb=histb):
                for u in range(16):
                    histb[pl.ds(z * 256 + u * 16, 16)] = zeros16
                return 0
            with jax.named_scope("sc_zero"):
                lax.fori_loop(0, CHWORDS // 256, zbody, 0)

            with jax.named_scope("sc_stagewait"):
                vdescs[c].wait()

            lo0 = offs[c] - vbase[c]

            def bag_group(g, o, histb=histb, valsb=valsb, lo0=lo0, c=c):
                lv = len_v[pl.ds(my_first + c * CBAGS + g * 16, 16)]
                amt = zeros16 + (jnp.int32(1) << ((g >> 1) * 8))
                gbase = (g & 1) * (16 * 128)
                for u in range(16):
                    ln = lv[u]
                    lo = o + lo0
                    jb = gbase + u * 128

                    def scat(v, m, jb=jb):
                        idx = ((v >> 7) * (CW * 128)) + jb + (v & 127)
                        plsc.addupdate_scatter(histb, [idx], amt, mask=m)

                    m0 = lane < ln
                    v0 = plsc.load_gather(valsb, [lo + lane], mask=m0)
                    scat(v0, m0 & (v0 < NV))

                    @pl.when(ln > 16)
                    def _(lo=lo, ln=ln, scat=scat):
                        m1 = lane + 16 < ln
                        v1 = plsc.load_gather(valsb, [lo + 16 + lane],
                                              mask=m1)
                        scat(v1, m1 & (v1 < NV))

                    o = o + ln
                return o

            with jax.named_scope("sc_bags"):
                lax.fori_loop(0, CBAGS // 16, bag_group, jnp.int32(0))

            if c + 2 < NCHUNK:
                _fire_stage(c + 2)

            row0 = my_first + c * CBAGS
            with jax.named_scope("sc_histout"):
                hdescs[c] = []
                for vt in range(NT):
                    hdescs[c].append(pltpu.async_copy(
                        histb.at[pl.ds(vt * CW * 128, CW * 128)],
                        hist_hbm.at[pl.ds(vt * (B // NPACK) * 128
                                          + (row0 // NPACK) * 128, CW * 128)],
                        hsems[buf]))

            if c + 1 < NCHUNK and c >= 1:
                with jax.named_scope("sc_histdrain"):
                    for d in hdescs[c - 1]:
                        d.wait()

        def tz(z, _):
            tacc_v[pl.ds(z * 16, 16)] = jnp.zeros((16,), jnp.float32)
            return 0
        lax.fori_loop(0, NV // 16, tz, 0)

        with jax.named_scope("sc_tail"):
            for o in range(NTO):
                tdescs[o].wait()
                st = start_w + o * TSTAGE
                shift = st - tst_al[o]
                tbuf = tstage_bufs[o % 2]

                @pl.when(o < n_out)
                def _(st=st, shift=shift, tbuf=tbuf):
                    def tinner(g, _):
                        for u in range(4):
                            pos = st + g * 64 + u * 16
                            mp = (pos + lane) < limit
                            vv = plsc.load_gather(
                                tbuf, [shift + g * 64 + u * 16 + lane],
                                mask=mp)
                            plsc.addupdate_scatter(tacc_v, [vv], onesf,
                                                   mask=mp & (vv < NV))
                        return 0

                    lax.fori_loop(0, TSTAGE // 64, tinner, 0)

                if o + 2 < NTO:
                    _fire_tail(o + 2)

            for vt in range(NT):
                pltpu.sync_copy(
                    tacc_v.at[pl.ds(vt * 128, 128)],
                    tail_hbm.at[pl.ds(vt * NW * 128 + wid * 128, 128)])

        with jax.named_scope("sc_finaldrain"):
            for c in (NCHUNK - 2, NCHUNK - 1):
                for d in hdescs[c]:
                    d.wait()

    return k(values_padded, lengths)


def _matmul(hist3p, w2, tail):
    BM = 1024
    BMP = BM // NPACK
    nb = B // BM

    def body(h_ref, w2_ref, t_ref, oa_ref, ob_ref, hf_ref):
        for j in range(NT):
            w = h_ref[j]
            bytes_f = [((w >> (8 * kk)) & 0xFF).astype(jnp.bfloat16)
                       for kk in range(NPACK)]
            for g in range(BM // 128):
                for kk in range(NPACK):
                    hf_ref[g * 128 + kk * 32:g * 128 + (kk + 1) * 32,
                           j * 128:(j + 1) * 128] = (
                        bytes_f[kk][g * 32:(g + 1) * 32])
        acc = jnp.dot(hf_ref[...], w2_ref[...],
                      preferred_element_type=jnp.float32)
        tcon = jnp.dot(
            jnp.sum(t_ref[0], axis=0, keepdims=True).astype(jnp.bfloat16),
            w2_ref[:128, :], preferred_element_type=jnp.float32)
        for j in range(1, NT):
            tcon = tcon + jnp.dot(
                jnp.sum(t_ref[j], axis=0, keepdims=True).astype(jnp.bfloat16),
                w2_ref[j * 128:(j + 1) * 128, :],
                preferred_element_type=jnp.float32)
        row = lax.broadcasted_iota(jnp.int32, (BM, 1), 0)
        sel = (row == BM - 1) & (pl.program_id(0) == nb - 1)
        acc = jnp.clip(acc + jnp.where(sel, tcon, 0.0), 0.0, 1.0)
        oa_ref[...] = acc[:, :DOUT]
        ob_ref[...] = acc[:, DOUT:]

    return pl.pallas_call(
        body,
        grid=(nb,),
        in_specs=[
            pl.BlockSpec((NT, BMP, 128), lambda i: (0, i, 0)),
            pl.BlockSpec((NV, 2 * DOUT), lambda i: (0, 0)),
            pl.BlockSpec((NT, NW, 128), lambda i: (0, 0, 0)),
        ],
        out_specs=[
            pl.BlockSpec((BM, DOUT), lambda i: (i, 0)),
            pl.BlockSpec((BM, DOUT), lambda i: (i, 0)),
        ],
        out_shape=[
            jax.ShapeDtypeStruct((B, DOUT), jnp.float32),
            jax.ShapeDtypeStruct((B, DOUT), jnp.float32),
        ],
        scratch_shapes=[pltpu.VMEM((BM, NV), jnp.bfloat16)],
        compiler_params=pltpu.CompilerParams(
            dimension_semantics=("parallel",)),
    )(hist3p, w2, tail)


def kernel(pieces, ranks, files, tiles, factorization_mask, values, lengths):
    w2 = _build_w2(pieces, ranks, files, tiles, factorization_mask)
    hist_flat, tail_flat = _sc_hist(values, lengths)
    hist3p = hist_flat.reshape(NT, B // NPACK, 128)
    tail3 = tail_flat.reshape(NT, NW, 128)
    return _matmul(hist3p, w2, tail3)
